# baseline (speedup 1.0000x reference)
import jax
import jax.numpy as jnp
from jax.experimental import pallas as pl
from jax.experimental.pallas import tpu as pltpu
from jax.experimental.pallas import tpu_sc as plsc

_BR = 16
_LANES = 16
_SB = 256
_SC_BATCH = 1


def _sc_body(x_vmem, pe_vmem, o_vmem):
    x2 = x_vmem.at[0]
    o2 = o_vmem.at[0]
    ncols = pe_vmem.shape[1]

    @pl.loop(0, _BR)
    def _(r):
        @plsc.parallel_loop(0, ncols, step=_LANES, unroll=8)
        def _(c):
            slc = (pl.ds(r, 1), pl.ds(c, _LANES))
            o2.at[*slc][...] = x2.at[*slc][...] + pe_vmem.at[*slc][...]


def _sc_part(x, pe_weight):
    B, S, D = x.shape

    @pl.kernel(
        out_type=jax.ShapeDtypeStruct((_SC_BATCH, S, D), x.dtype),
        mesh=plsc.VectorSubcoreMesh(core_axis_name="c", subcore_axis_name="s"),
        compiler_params=pltpu.CompilerParams(use_tc_tiling_on_sc=True),
    )
    def run(x_hbm, pe_hbm, o_hbm):
        pltpu.emit_pipeline(
            _sc_body,
            grid=(_SC_BATCH, S // _BR),
            in_specs=[
                pl.BlockSpec((1, _BR, D), lambda b, i: (b, i, 0)),
                pl.BlockSpec((_BR, D), lambda b, i: (i, 0)),
            ],
            out_specs=[pl.BlockSpec((1, _BR, D), lambda b, i: (b, i, 0))],
            core_axis_name=("c", "s"),
            dimension_semantics=(pltpu.PARALLEL, pltpu.PARALLEL),
        )(x_hbm, pe_hbm, o_hbm)

    return run(x, pe_weight)


def _tc_add(x_ref, pe_ref, o_ref):
    o_ref[...] = x_ref[...] + pe_ref[...]


def _tc_part(x, pe_weight):
    B, S, D = x.shape
    nb = B - _SC_BATCH
    return pl.pallas_call(
        _tc_add,
        grid=(S // _SB, nb),
        in_specs=[
            pl.BlockSpec((1, _SB, D), lambda s, b: (b + _SC_BATCH, s, 0)),
            pl.BlockSpec((_SB, D), lambda s, b: (s, 0)),
        ],
        out_specs=pl.BlockSpec((1, _SB, D), lambda s, b: (b, s, 0)),
        out_shape=jax.ShapeDtypeStruct((nb, S, D), x.dtype),
    )(x, pe_weight)


def kernel(x, pe_weight):
    out_sc = _sc_part(x, pe_weight)
    out_tc = _tc_part(x, pe_weight)
    return jnp.concatenate([out_sc, out_tc], axis=0)


# speedup vs baseline: 2.8837x; 2.8837x over previous
import jax
import jax.numpy as jnp
from jax.experimental import pallas as pl


def _add_kernel(x_ref, pe_ref, o_ref):
    o_ref[...] = x_ref[...] + pe_ref[...]


def kernel(x, pe_weight):
    B, S, D = x.shape
    SB = 512
    return pl.pallas_call(
        _add_kernel,
        grid=(S // SB,),
        in_specs=[
            pl.BlockSpec((B, SB, D), lambda s: (0, s, 0)),
            pl.BlockSpec((SB, D), lambda s: (s, 0)),
        ],
        out_specs=pl.BlockSpec((B, SB, D), lambda s: (0, s, 0)),
        out_shape=jax.ShapeDtypeStruct((B, S, D), x.dtype),
    )(x, pe_weight)
